# Initial kernel scaffold; baseline (speedup 1.0000x reference)
import jax
import jax.numpy as jnp
from jax.experimental import pallas as pl


def kernel(user_emb, item_emb, edge_weight, edge_index):
    raise NotImplementedError("write your pallas kernel here")


# speedup vs baseline: 6.5621x; 6.5621x over previous
import functools

import jax
import jax.numpy as jnp
from jax import lax
from jax.experimental import pallas as pl
from jax.experimental.pallas import tpu as pltpu
from jax.experimental.pallas import tpu_sc as plsc

NUM_USERS = 20000
NUM_ITEMS = 30000
N_NODES = NUM_USERS + NUM_ITEMS
N_EDGES = 800000
LATENT_DIM = 64
HALF_DIM = 32
N_LAYERS = 3

NC = 2
NS = 16

CHUNK = 512
SUB = 128
NSUB = CHUNK // SUB
NSUP = 98
EDGES_PER_TILE = NSUP * CHUNK
E_PAD = EDGES_PER_TILE * NS
PLANE = 51200
ROWS_PER_TILE = PLANE // NS


def _layer_body(emb, src, w, dst2d, out, rows, srcv, gidx, wv, didx, acc, sem):
    c = lax.axis_index("c")
    s = lax.axis_index("s")
    plane = c * PLANE

    zeros16 = jnp.zeros((16,), jnp.float32)

    def zero_rows(i, carry):
        rows[i, pl.ds(0, 16)] = zeros16
        rows[i, pl.ds(16, 16)] = zeros16
        return carry

    lax.fori_loop(0, CHUNK, zero_rows, 0)

    rbase = s * ROWS_PER_TILE
    for k in range(6):
        pltpu.sync_copy(rows.at[pl.ds(0, 512)],
                        acc.at[pl.ds(rbase + k * 512, 512)])
    pltpu.sync_copy(rows.at[pl.ds(0, 128)], acc.at[pl.ds(rbase + 3072, 128)])
    plsc.subcore_barrier()

    def superchunk(j, carry):
        base_e = s * EDGES_PER_TILE + j * CHUNK
        base_r = s * (EDGES_PER_TILE // SUB) + j * NSUB
        pltpu.sync_copy(src.at[pl.ds(base_e, CHUNK)], srcv)
        pltpu.sync_copy(w.at[pl.ds(base_e, CHUNK)], wv)
        pltpu.sync_copy(dst2d.at[pl.ds(base_r, NSUB)], didx)

        def make_gidx(g, carry2):
            gidx[pl.ds(g * 16, 16)] = srcv[pl.ds(g * 16, 16)] + plane
            return carry2

        lax.fori_loop(0, CHUNK // 16, make_gidx, 0)

        copies = [
            pltpu.async_copy(emb.at[gidx.at[pl.ds(b * SUB, SUB)]],
                             rows.at[pl.ds(b * SUB, SUB)], sem)
            for b in range(NSUB)
        ]
        for cp in copies:
            cp.wait()

        def scale(g, carry2):
            wv16 = wv[pl.ds(g * 16, 16)]
            for l in range(16):
                e = g * 16 + l
                wsc = wv16[l]
                rows[e, pl.ds(0, 16)] = rows[e, pl.ds(0, 16)] * wsc
                rows[e, pl.ds(16, 16)] = rows[e, pl.ds(16, 16)] * wsc
            return carry2

        lax.fori_loop(0, CHUNK // 16, scale, 0)

        for b in range(NSUB):
            pltpu.sync_copy(rows.at[pl.ds(b * SUB, SUB)],
                            acc.at[didx.at[b]], add=True)
        return carry

    lax.fori_loop(0, NSUP, superchunk, 0)

    plsc.subcore_barrier()
    pltpu.sync_copy(acc.at[pl.ds(rbase, ROWS_PER_TILE)],
                    out.at[pl.ds(plane + rbase, ROWS_PER_TILE)])


_layer = pl.kernel(
    _layer_body,
    out_type=jax.ShapeDtypeStruct((NC * PLANE, HALF_DIM), jnp.float32),
    mesh=plsc.VectorSubcoreMesh(core_axis_name="c", subcore_axis_name="s",
                                num_cores=NC, num_subcores=NS),
    scratch_types=[
        pltpu.VMEM((CHUNK, HALF_DIM), jnp.float32),
        pltpu.VMEM((CHUNK,), jnp.int32),
        pltpu.VMEM((CHUNK,), jnp.int32),
        pltpu.VMEM((CHUNK,), jnp.float32),
        pltpu.VMEM((NSUB, SUB), jnp.int32),
        pltpu.VMEM_SHARED((PLANE, HALF_DIM), jnp.float32),
        pltpu.SemaphoreType.DMA,
    ],
    compiler_params=pltpu.CompilerParams(use_tc_tiling_on_sc=False),
)


def _mean_body(t0, t1, t2, t3, b0, b1, b2, b3, o):
    top = t0[...] + t1[...] + t2[...] + t3[...]
    bot = b0[...] + b1[...] + b2[...] + b3[...]
    o[...] = jnp.concatenate([top, bot], axis=1) * 0.25


_MEAN_BLOCK = 400
_N_BLOCKS = N_NODES // _MEAN_BLOCK
_PLANE_BLOCKS = PLANE // _MEAN_BLOCK


def _mean4(e0, e1, e2, e3):
    top_spec = pl.BlockSpec((_MEAN_BLOCK, HALF_DIM), lambda i: (i, 0))
    bot_spec = pl.BlockSpec((_MEAN_BLOCK, HALF_DIM),
                            lambda i: (i + _PLANE_BLOCKS, 0))
    return pl.pallas_call(
        _mean_body,
        grid=(_N_BLOCKS,),
        in_specs=[top_spec] * 4 + [bot_spec] * 4,
        out_specs=pl.BlockSpec((_MEAN_BLOCK, LATENT_DIM), lambda i: (i, 0)),
        out_shape=jax.ShapeDtypeStruct((N_NODES, LATENT_DIM), jnp.float32),
    )(e0, e1, e2, e3, e0, e1, e2, e3)


@jax.jit
def kernel(user_emb, item_emb, edge_weight, edge_index):
    src = edge_index[0].astype(jnp.int32)
    dst = edge_index[1].astype(jnp.int32)
    w = edge_weight.astype(jnp.float32)
    pad = E_PAD - N_EDGES
    src = jnp.pad(src, (0, pad))
    dstp = jnp.pad(dst, (0, pad))
    wp = jnp.pad(w, (0, pad))
    dst2d = dstp.reshape(E_PAD // SUB, SUB)

    e0 = jnp.concatenate([user_emb, item_emb], axis=0)
    planes = e0.reshape(N_NODES, NC, HALF_DIM).transpose(1, 0, 2)
    planes = jnp.pad(planes, ((0, 0), (0, PLANE - N_NODES), (0, 0)))
    e0s = planes.reshape(NC * PLANE, HALF_DIM)

    e1 = _layer(e0s, src, wp, dst2d)
    e2 = _layer(e1, src, wp, dst2d)
    e3 = _layer(e2, src, wp, dst2d)

    out = _mean4(e0s, e1, e2, e3)
    return out[:NUM_USERS], out[NUM_USERS:]
